# Initial kernel scaffold; baseline (speedup 1.0000x reference)
import jax
import jax.numpy as jnp
from jax.experimental import pallas as pl


def kernel(n_bar, table):
    raise NotImplementedError("write your pallas kernel here")


# speedup vs baseline: 1.9447x; 1.9447x over previous
import functools

import jax
import jax.numpy as jnp
from jax import lax
from jax.experimental import pallas as pl
from jax.experimental.pallas import tpu as pltpu
from jax.experimental.pallas import tpu_sc as plsc

MAX_BAR = 128
LEN_EMBED_DIM = 128
LENGTH_BUCKET_SIZE = 10
BATCH = 16384

_INFO = plsc.get_sparse_core_info()
_NC, _NS = _INFO.num_cores, _INFO.num_subcores
_NW = _NC * _NS
_BPW = BATCH // _NW
_NSTREAM = _BPW // 128


def _sc_body(nbar_hbm, table_hbm, out_hbm, nbar_v, idx_v, rows_v, sem):
    wid = lax.axis_index("s") * _NC + lax.axis_index("c")
    base = wid * _BPW
    pltpu.sync_copy(nbar_hbm.at[pl.ds(base, _BPW)], nbar_v)
    inv = jnp.float32(LENGTH_BUCKET_SIZE)
    for i in range(_BPW // 16):
        v = nbar_v[pl.ds(i * 16, 16)]
        b = (v.astype(jnp.float32) / inv).astype(jnp.int32)
        idx_v[i // 8, pl.ds((i % 8) * 16, 16)] = b
    copies = [
        pltpu.async_copy(
            table_hbm.at[idx_v.at[j]],
            rows_v.at[pl.ds(j * 128, 128)],
            sem,
        )
        for j in range(_NSTREAM)
    ]
    for c in copies:
        c.wait()
    pltpu.sync_copy(rows_v, out_hbm.at[pl.ds(base, _BPW)])


@jax.jit
def kernel(n_bar, table):
    n_bar = n_bar.astype(jnp.int32)
    mesh = plsc.VectorSubcoreMesh(core_axis_name="c", subcore_axis_name="s")
    out = pl.kernel(
        _sc_body,
        mesh=mesh,
        out_type=jax.ShapeDtypeStruct((BATCH, LEN_EMBED_DIM), jnp.float32),
        scratch_types=[
            pltpu.VMEM((_BPW,), jnp.int32),
            pltpu.VMEM((_NSTREAM, 128), jnp.int32),
            pltpu.VMEM((_BPW, LEN_EMBED_DIM), jnp.float32),
            pltpu.SemaphoreType.DMA,
        ],
    )(n_bar, table)
    return out[:, None, :]
